# baseline (speedup 1.0000x reference)
import jax
import jax.numpy as jnp
from jax.experimental import pallas as pl

_B, _N, _C = 128, 25, 128
_J = 25
_HID = 32
_K = 8
_E = _J * _J


_U = 32


def _edge_attn_body(spdt_ref, srct_ref, w1f_ref, cpos_ref, cneg_ref, out_ref):
    spdt = spdt_ref[...]
    n_iota = jax.lax.broadcasted_iota(jnp.int32, (_N, _E), 0)
    dt = jnp.zeros((_N, _E), dtype=jnp.float32)
    for k in range(_K - 1):
        ends_k = spdt[k][None, :]
        heads_k = spdt[k][None, :]
        dt = dt + (ends_k == n_iota).astype(jnp.float32)
        dt = dt - (heads_k == n_iota).astype(jnp.float32)

    w1f = w1f_ref[...]
    cpos = cpos_ref[...]
    cneg = cneg_ref[...]

    def per_group(i, carry):
        base = i * _U
        pts = [
            jnp.dot(w1f, srct_ref[base + j], preferred_element_type=jnp.float32)
            for j in range(_U)
        ]
        pcat = jnp.concatenate(pts, axis=0)
        hcat = jnp.dot(pcat, dt, preferred_element_type=jnp.float32)
        g = cpos * jnp.maximum(hcat, 0.0) + cneg * jnp.minimum(hcat, 0.0)
        og = jnp.sum(g.reshape(_U, _HID, _E), axis=1)
        out_ref[pl.ds(base, _U), :] = og
        return carry

    jax.lax.fori_loop(0, _B // _U, per_group, 0)


def kernel(src, s_SPD, W1, a, W2):
    spdt = s_SPD.reshape(_E, _K).T
    srct = src.transpose(0, 2, 1)
    w2row = W2[0]
    w1f = W1 * w2row[:, None]
    cpos1 = jnp.where(w2row >= 0, 1.0, a[0]).astype(jnp.float32)
    cneg1 = jnp.where(w2row >= 0, a[0], 1.0).astype(jnp.float32)
    cpos = jnp.tile(cpos1, _U)[:, None]
    cneg = jnp.tile(cneg1, _U)[:, None]
    out = pl.pallas_call(
        _edge_attn_body,
        in_specs=[
            pl.BlockSpec((_K, _E), lambda: (0, 0)),
            pl.BlockSpec((_B, _C, _N), lambda: (0, 0, 0)),
            pl.BlockSpec((_HID, _C), lambda: (0, 0)),
            pl.BlockSpec((_U * _HID, 1), lambda: (0, 0)),
            pl.BlockSpec((_U * _HID, 1), lambda: (0, 0)),
        ],
        out_specs=pl.BlockSpec((_B, _E), lambda: (0, 0)),
        out_shape=jax.ShapeDtypeStruct((_B, _E), jnp.float32),
    )(spdt, srct, w1f, cpos, cneg)
    return out.reshape(_B, _J, _J, 1)


# speedup vs baseline: 1.5717x; 1.5717x over previous
import jax
import jax.numpy as jnp
from jax.experimental import pallas as pl

_B, _N, _C = 128, 25, 128
_J = 25
_HID = 32
_K = 8
_E = _J * _J
_U = 32


def _edge_attn_body(spd_ref, src_ref, w1_ref, a_ref, w2_ref, out_ref):
    spd = spd_ref[...]
    n_iota = jax.lax.broadcasted_iota(jnp.int32, (_E, _N), 1)
    d = jnp.zeros((_E, _N), dtype=jnp.float32)
    for k in range(_K - 1):
        ends_k = spd[:, k][:, None]
        heads_k = spd[:, k][:, None]
        d = d + (ends_k == n_iota).astype(jnp.float32)
        d = d - (heads_k == n_iota).astype(jnp.float32)
    dt = d.T

    alpha = a_ref[0, 0]
    w2col = w2_ref[...].T
    w1f = w1_ref[...] * w2col
    cpos1 = jnp.where(w2col >= 0, 1.0, alpha)
    cneg1 = jnp.where(w2col >= 0, alpha, 1.0)
    cpos = jnp.concatenate([cpos1] * _U, axis=0)
    cneg = jnp.concatenate([cneg1] * _U, axis=0)

    def per_group(i, carry):
        base = i * _U
        pts = [
            jnp.dot(w1f, src_ref[base + j].T, preferred_element_type=jnp.float32)
            for j in range(_U)
        ]
        pcat = jnp.concatenate(pts, axis=0)
        hcat = jnp.dot(pcat, dt, preferred_element_type=jnp.float32)
        g = cpos * jnp.maximum(hcat, 0.0) + cneg * jnp.minimum(hcat, 0.0)
        og = jnp.sum(g.reshape(_U, _HID, _E), axis=1)
        out_ref[pl.ds(base, _U), :] = og
        return carry

    jax.lax.fori_loop(0, _B // _U, per_group, 0)


def kernel(src, s_SPD, W1, a, W2):
    spd = s_SPD.reshape(_E, _K)
    a2 = a.reshape(1, 1)
    out = pl.pallas_call(
        _edge_attn_body,
        in_specs=[
            pl.BlockSpec((_E, _K), lambda: (0, 0)),
            pl.BlockSpec((_B, _N, _C), lambda: (0, 0, 0)),
            pl.BlockSpec((_HID, _C), lambda: (0, 0)),
            pl.BlockSpec((1, 1), lambda: (0, 0)),
            pl.BlockSpec((1, _HID), lambda: (0, 0)),
        ],
        out_specs=pl.BlockSpec((_B, _E), lambda: (0, 0)),
        out_shape=jax.ShapeDtypeStruct((_B, _E), jnp.float32),
    )(spd, src, W1, a2, W2)
    return out.reshape(_B, _J, _J, 1)
